# baseline (speedup 1.0000x reference)
import functools

import jax
import jax.numpy as jnp
from jax import lax
from jax.experimental import pallas as pl
from jax.experimental.pallas import tpu as pltpu
from jax.experimental.pallas import tpu_sc as plsc

B, T, V = 16, 16, 100000
C, K = 4, 8
NUM_TOK = C * K


def _gather_sc(table_flat, tok_flat):
    info = plsc.get_sparse_core_info()
    nc, ns = info.num_cores, info.num_subcores
    nw = nc * ns
    rows = B * T
    rows_per_w = rows // nw
    per_w = rows_per_w * NUM_TOK

    mesh = plsc.VectorSubcoreMesh(core_axis_name="c", subcore_axis_name="s")

    @functools.partial(
        pl.kernel,
        mesh=mesh,
        out_type=jax.ShapeDtypeStruct((rows * NUM_TOK,), jnp.float32),
        scratch_types=[
            pltpu.VMEM((NUM_TOK,), jnp.int32),
            pltpu.VMEM((per_w,), jnp.int32),
            pltpu.VMEM((per_w,), jnp.float32),
            pltpu.SemaphoreType.DMA,
        ],
    )
    def k(table_hbm, tok_hbm, out_hbm, tok_v, idx_v, rows_v, sem):
        wid = lax.axis_index("s") * nc + lax.axis_index("c")
        pltpu.sync_copy(tok_hbm, tok_v)
        base_row = wid * rows_per_w
        for r in range(rows_per_w):
            off = (base_row + r) * V
            for h in range(NUM_TOK // 16):
                chunk = tok_v[pl.ds(h * 16, 16)]
                idx_v[pl.ds(r * NUM_TOK + h * 16, 16)] = chunk + off
        pltpu.async_copy(table_hbm.at[idx_v], rows_v, sem).wait()
        pltpu.sync_copy(rows_v, out_hbm.at[pl.ds(wid * per_w, per_w)])

    return k(table_flat, tok_flat)


def _softmax_combine_body(x_ref, g_ref, o_ref):
    x = x_ref[0]
    m = jnp.max(x, axis=1, keepdims=True)
    denom = jnp.sum(jnp.exp(x - m), axis=1, keepdims=True)
    p = jnp.exp(g_ref[0] - m) / denom
    sel = (
        lax.broadcasted_iota(jnp.int32, (NUM_TOK, C), 0) // K
        == lax.broadcasted_iota(jnp.int32, (NUM_TOK, C), 1)
    ).astype(jnp.float32)
    acc = jnp.dot(p, sel, preferred_element_type=jnp.float32)
    o_ref[0, 0] = jnp.sum(acc, axis=0) * (1.0 / (T * K))


def _softmax_combine(lm_logits, g, interpret=False):
    out = pl.pallas_call(
        _softmax_combine_body,
        grid=(B,),
        in_specs=[
            pl.BlockSpec((1, T, V), lambda b: (b, 0, 0)),
            pl.BlockSpec((1, T, NUM_TOK), lambda b: (b, 0, 0)),
        ],
        out_specs=pl.BlockSpec((1, 1, C), lambda b: (b, 0, 0)),
        out_shape=jax.ShapeDtypeStruct((B, 1, C), jnp.float32),
        interpret=interpret,
    )(lm_logits, g)
    return out.reshape(B, C)


def kernel(lm_logits, token_ids):
    tok_flat = token_ids.reshape(-1)
    g = _gather_sc(lm_logits.reshape(-1), tok_flat).reshape(B, T, NUM_TOK)
    return _softmax_combine(lm_logits, g)
